# baseline (speedup 1.0000x reference)
import functools

import jax
import jax.numpy as jnp
from jax import lax
from jax.experimental import pallas as pl
from jax.experimental.pallas import tpu as pltpu
from jax.experimental.pallas import tpu_sc as plsc

VOCAB = 1000000
DIM = 64
BATCH = 4096
SEQ = 200
NUM_CLASSES = 2

_NUM_WORKERS = 32
_ROWS_PER_W = BATCH // _NUM_WORKERS
_HALF = SEQ // 2


def _pool_kernel(x_hbm, embed_hbm, out_hbm, idx_v, buf_v, out_v, sem):
    wid = lax.axis_index("s") * 2 + lax.axis_index("c")
    base = wid * _ROWS_PER_W

    pltpu.sync_copy(x_hbm.at[pl.ds(base, _ROWS_PER_W)], idx_v)

    def row_body(r, _):
        c0 = pltpu.async_copy(embed_hbm.at[idx_v.at[r, 0]], buf_v.at[0], sem)
        c1 = pltpu.async_copy(embed_hbm.at[idx_v.at[r, 1]], buf_v.at[1], sem)
        c0.wait()
        c1.wait()

        def acc_body(j, accs):
            a0, a1, a2, a3 = accs
            a0 = a0 + buf_v[0, j, pl.ds(0, 16)] + buf_v[1, j, pl.ds(0, 16)]
            a1 = a1 + buf_v[0, j, pl.ds(16, 16)] + buf_v[1, j, pl.ds(16, 16)]
            a2 = a2 + buf_v[0, j, pl.ds(32, 16)] + buf_v[1, j, pl.ds(32, 16)]
            a3 = a3 + buf_v[0, j, pl.ds(48, 16)] + buf_v[1, j, pl.ds(48, 16)]
            return (a0, a1, a2, a3)

        zero = jnp.zeros((16,), jnp.float32)
        a0, a1, a2, a3 = lax.fori_loop(0, _HALF, acc_body,
                                       (zero, zero, zero, zero))
        scale = jnp.float32(1.0 / SEQ)
        out_v[r, pl.ds(0, 16)] = a0 * scale
        out_v[r, pl.ds(16, 16)] = a1 * scale
        out_v[r, pl.ds(32, 16)] = a2 * scale
        out_v[r, pl.ds(48, 16)] = a3 * scale
        return 0

    lax.fori_loop(0, _ROWS_PER_W, row_body, 0)

    pltpu.sync_copy(out_v, out_hbm.at[pl.ds(base, _ROWS_PER_W)])


@functools.partial(
    pl.kernel,
    mesh=plsc.VectorSubcoreMesh(core_axis_name="c", subcore_axis_name="s"),
    out_type=jax.ShapeDtypeStruct((BATCH, DIM), jnp.float32),
    scratch_types=[
        pltpu.VMEM((_ROWS_PER_W, 2, _HALF), jnp.int32),
        pltpu.VMEM((2, _HALF, DIM), jnp.float32),
        pltpu.VMEM((_ROWS_PER_W, DIM), jnp.float32),
        pltpu.SemaphoreType.DMA,
    ],
    compiler_params=pltpu.CompilerParams(use_tc_tiling_on_sc=False),
)
def _pooled(x_hbm, embed_hbm, out_hbm, idx_v, buf_v, out_v, sem):
    _pool_kernel(x_hbm, embed_hbm, out_hbm, idx_v, buf_v, out_v, sem)


def _mm_kernel(p_ref, w_ref, b_ref, o_ref):
    o_ref[...] = jnp.dot(p_ref[...], w_ref[...],
                         preferred_element_type=jnp.float32) + b_ref[...]


def kernel(x, embed, fc1_w, fc1_b):
    x32 = x.astype(jnp.int32).reshape(BATCH, 2, _HALF)
    pooled = _pooled(x32, embed)
    logits = pl.pallas_call(
        _mm_kernel,
        out_shape=jax.ShapeDtypeStruct((BATCH, NUM_CLASSES), jnp.float32),
    )(pooled, fc1_w.T, fc1_b.reshape(1, NUM_CLASSES))
    return logits


# speedup vs baseline: 1.1325x; 1.1325x over previous
import functools

import jax
import jax.numpy as jnp
from jax import lax
from jax.experimental import pallas as pl
from jax.experimental.pallas import tpu as pltpu
from jax.experimental.pallas import tpu_sc as plsc

VOCAB = 1000000
DIM = 64
BATCH = 4096
SEQ = 200
NUM_CLASSES = 2

_NUM_WORKERS = 32
_ROWS_PER_W = BATCH // _NUM_WORKERS
_HALF = SEQ // 2


def _pool_kernel(x_hbm, embed_hbm, out_hbm, idx_v, buf_v, out_v, sem0, sem1):
    wid = lax.axis_index("s") * 2 + lax.axis_index("c")
    base = wid * _ROWS_PER_W

    pltpu.sync_copy(x_hbm.at[pl.ds(base, _ROWS_PER_W)], idx_v)

    sems = (sem0, sem1)

    def issue(r, slot):
        pltpu.async_copy(embed_hbm.at[idx_v.at[r, 0]], buf_v.at[slot, 0],
                         sems[slot])
        pltpu.async_copy(embed_hbm.at[idx_v.at[r, 1]], buf_v.at[slot, 1],
                         sems[slot])

    def drain(r, slot):
        pltpu.make_async_copy(embed_hbm.at[idx_v.at[r, 0]],
                              buf_v.at[slot, 0], sems[slot]).wait()
        pltpu.make_async_copy(embed_hbm.at[idx_v.at[r, 1]],
                              buf_v.at[slot, 1], sems[slot]).wait()

    def accumulate(r, slot):
        def acc_body(j, accs):
            a0, a1, a2, a3 = accs
            a0 = (a0 + buf_v[slot, 0, j, pl.ds(0, 16)]
                  + buf_v[slot, 1, j, pl.ds(0, 16)])
            a1 = (a1 + buf_v[slot, 0, j, pl.ds(16, 16)]
                  + buf_v[slot, 1, j, pl.ds(16, 16)])
            a2 = (a2 + buf_v[slot, 0, j, pl.ds(32, 16)]
                  + buf_v[slot, 1, j, pl.ds(32, 16)])
            a3 = (a3 + buf_v[slot, 0, j, pl.ds(48, 16)]
                  + buf_v[slot, 1, j, pl.ds(48, 16)])
            return (a0, a1, a2, a3)

        zero = jnp.zeros((16,), jnp.float32)
        a0, a1, a2, a3 = lax.fori_loop(0, _HALF, acc_body,
                                       (zero, zero, zero, zero), unroll=10)
        scale = jnp.float32(1.0 / SEQ)
        out_v[r, pl.ds(0, 16)] = a0 * scale
        out_v[r, pl.ds(16, 16)] = a1 * scale
        out_v[r, pl.ds(32, 16)] = a2 * scale
        out_v[r, pl.ds(48, 16)] = a3 * scale

    issue(0, 0)

    def pair_body(t, _):
        r0 = 2 * t
        issue(r0 + 1, 1)
        drain(r0, 0)
        accumulate(r0, 0)

        @pl.when(t < _ROWS_PER_W // 2 - 1)
        def _():
            issue(r0 + 2, 0)

        drain(r0 + 1, 1)
        accumulate(r0 + 1, 1)
        return 0

    lax.fori_loop(0, _ROWS_PER_W // 2, pair_body, 0)

    pltpu.sync_copy(out_v, out_hbm.at[pl.ds(base, _ROWS_PER_W)])


@functools.partial(
    pl.kernel,
    mesh=plsc.VectorSubcoreMesh(core_axis_name="c", subcore_axis_name="s"),
    out_type=jax.ShapeDtypeStruct((BATCH, DIM), jnp.float32),
    scratch_types=[
        pltpu.VMEM((_ROWS_PER_W, 2, _HALF), jnp.int32),
        pltpu.VMEM((2, 2, _HALF, DIM), jnp.float32),
        pltpu.VMEM((_ROWS_PER_W, DIM), jnp.float32),
        pltpu.SemaphoreType.DMA,
        pltpu.SemaphoreType.DMA,
    ],
    compiler_params=pltpu.CompilerParams(use_tc_tiling_on_sc=False),
)
def _pooled(x_hbm, embed_hbm, out_hbm, idx_v, buf_v, out_v, sem0, sem1):
    _pool_kernel(x_hbm, embed_hbm, out_hbm, idx_v, buf_v, out_v, sem0, sem1)


def _mm_kernel(p_ref, w_ref, b_ref, o_ref):
    o_ref[...] = jnp.dot(p_ref[...], w_ref[...],
                         preferred_element_type=jnp.float32) + b_ref[...]


def kernel(x, embed, fc1_w, fc1_b):
    x32 = x.astype(jnp.int32).reshape(BATCH, 2, _HALF)
    pooled = _pooled(x32, embed)
    logits = pl.pallas_call(
        _mm_kernel,
        out_shape=jax.ShapeDtypeStruct((BATCH, NUM_CLASSES), jnp.float32),
    )(pooled, fc1_w.T, fc1_b.reshape(1, NUM_CLASSES))
    return logits


# speedup vs baseline: 1.1368x; 1.0038x over previous
import functools

import jax
import jax.numpy as jnp
from jax import lax
from jax.experimental import pallas as pl
from jax.experimental.pallas import tpu as pltpu
from jax.experimental.pallas import tpu_sc as plsc

VOCAB = 1000000
DIM = 64
BATCH = 4096
SEQ = 200
NUM_CLASSES = 2

_NUM_WORKERS = 32
_ROWS_PER_W = BATCH // _NUM_WORKERS
_C0 = 96
_C1 = SEQ - _C0
_HALF = SEQ // 2


def _pool_kernel(x_hbm, embed_hbm, out_hbm, idx_v, buf_v, out_v, sem0, sem1):
    wid = lax.axis_index("s") * 2 + lax.axis_index("c")
    base = wid * _ROWS_PER_W

    pltpu.sync_copy(x_hbm.at[pl.ds(base, _ROWS_PER_W)], idx_v)

    sems = (sem0, sem1)

    def issue(r, slot):
        pltpu.async_copy(embed_hbm.at[idx_v.at[r, pl.ds(0, _C0)]],
                         buf_v.at[slot, pl.ds(0, _C0)], sems[slot])
        pltpu.async_copy(embed_hbm.at[idx_v.at[r, pl.ds(_C0, _C1)]],
                         buf_v.at[slot, pl.ds(_C0, _C1)], sems[slot])

    def drain(r, slot):
        pltpu.make_async_copy(embed_hbm.at[idx_v.at[r, pl.ds(0, _C0)]],
                              buf_v.at[slot, pl.ds(0, _C0)],
                              sems[slot]).wait()
        pltpu.make_async_copy(embed_hbm.at[idx_v.at[r, pl.ds(_C0, _C1)]],
                              buf_v.at[slot, pl.ds(_C0, _C1)],
                              sems[slot]).wait()

    def accumulate(r, slot):
        def acc_body(j, accs):
            a0, a1, a2, a3 = accs
            a0 = (a0 + buf_v[slot, j, pl.ds(0, 16)]
                  + buf_v[slot, j + _HALF, pl.ds(0, 16)])
            a1 = (a1 + buf_v[slot, j, pl.ds(16, 16)]
                  + buf_v[slot, j + _HALF, pl.ds(16, 16)])
            a2 = (a2 + buf_v[slot, j, pl.ds(32, 16)]
                  + buf_v[slot, j + _HALF, pl.ds(32, 16)])
            a3 = (a3 + buf_v[slot, j, pl.ds(48, 16)]
                  + buf_v[slot, j + _HALF, pl.ds(48, 16)])
            return (a0, a1, a2, a3)

        zero = jnp.zeros((16,), jnp.float32)
        a0, a1, a2, a3 = lax.fori_loop(0, _HALF, acc_body,
                                       (zero, zero, zero, zero), unroll=10)
        scale = jnp.float32(1.0 / SEQ)
        out_v[r, pl.ds(0, 16)] = a0 * scale
        out_v[r, pl.ds(16, 16)] = a1 * scale
        out_v[r, pl.ds(32, 16)] = a2 * scale
        out_v[r, pl.ds(48, 16)] = a3 * scale

    issue(0, 0)

    def pair_body(t, _):
        r0 = 2 * t
        issue(r0 + 1, 1)
        drain(r0, 0)
        accumulate(r0, 0)

        @pl.when(t < _ROWS_PER_W // 2 - 1)
        def _():
            issue(r0 + 2, 0)

        drain(r0 + 1, 1)
        accumulate(r0 + 1, 1)
        return 0

    lax.fori_loop(0, _ROWS_PER_W // 2, pair_body, 0)

    pltpu.sync_copy(out_v, out_hbm.at[pl.ds(base, _ROWS_PER_W)])


@functools.partial(
    pl.kernel,
    mesh=plsc.VectorSubcoreMesh(core_axis_name="c", subcore_axis_name="s"),
    out_type=jax.ShapeDtypeStruct((BATCH, DIM), jnp.float32),
    scratch_types=[
        pltpu.VMEM((_ROWS_PER_W, SEQ), jnp.int32),
        pltpu.VMEM((2, SEQ, DIM), jnp.float32),
        pltpu.VMEM((_ROWS_PER_W, DIM), jnp.float32),
        pltpu.SemaphoreType.DMA,
        pltpu.SemaphoreType.DMA,
    ],
    compiler_params=pltpu.CompilerParams(use_tc_tiling_on_sc=False),
)
def _pooled(x_hbm, embed_hbm, out_hbm, idx_v, buf_v, out_v, sem0, sem1):
    _pool_kernel(x_hbm, embed_hbm, out_hbm, idx_v, buf_v, out_v, sem0, sem1)


def _mm_kernel(p_ref, w_ref, b_ref, o_ref):
    o_ref[...] = jnp.dot(p_ref[...], w_ref[...],
                         preferred_element_type=jnp.float32) + b_ref[...]


def kernel(x, embed, fc1_w, fc1_b):
    x32 = x.astype(jnp.int32)
    pooled = _pooled(x32, embed)
    logits = pl.pallas_call(
        _mm_kernel,
        out_shape=jax.ShapeDtypeStruct((BATCH, NUM_CLASSES), jnp.float32),
    )(pooled, fc1_w.T, fc1_b.reshape(1, NUM_CLASSES))
    return logits
